# Initial kernel scaffold; baseline (speedup 1.0000x reference)
import jax
import jax.numpy as jnp
from jax.experimental import pallas as pl


def kernel(input_x, weight):
    raise NotImplementedError("write your pallas kernel here")


# speedup vs baseline: 1.8317x; 1.8317x over previous
import functools

import jax
import jax.numpy as jnp
from jax import lax
from jax.experimental import pallas as pl
from jax.experimental.pallas import tpu as pltpu
from jax.experimental.pallas import tpu_sc as plsc

_WORD_NUM = 1000000
_EMBED_DIM = 64
_BATCH = 16384
_HIST = 50
_B = _BATCH * _HIST

_info = plsc.get_sparse_core_info()
_NC = _info.num_cores
_NS = _info.num_subcores
_NW = _NC * _NS
_BPW = _B // _NW
_C = 512
_G = _BPW // _C

_mesh = plsc.VectorSubcoreMesh(core_axis_name="c", subcore_axis_name="s")


@functools.partial(
    pl.kernel,
    mesh=_mesh,
    out_type=jax.ShapeDtypeStruct((_B, _EMBED_DIM), jnp.float32),
    compiler_params=pltpu.CompilerParams(use_tc_tiling_on_sc=False),
    scratch_types=[
        pltpu.VMEM((_G, _C), jnp.int32),
        pltpu.VMEM((_C, _EMBED_DIM), jnp.float32),
        pltpu.SemaphoreType.DMA,
    ],
)
def _sc_gather(table_hbm, idx_hbm, out_hbm, idx_v, rows_v, sem):
    wid = lax.axis_index("s") * _NC + lax.axis_index("c")
    base = wid * _BPW
    pltpu.sync_copy(idx_hbm.at[wid], idx_v)

    def body(g, carry):
        pltpu.async_copy(table_hbm.at[idx_v.at[g]], rows_v, sem).wait()
        pltpu.sync_copy(rows_v, out_hbm.at[pl.ds(base + g * _C, _C)])
        return carry

    lax.fori_loop(0, _G, body, 0)


def kernel(input_x, weight):
    idx = input_x.reshape(_NW, _G, _C).astype(jnp.int32)
    out = _sc_gather(weight, idx)
    return out.reshape(_BATCH, _HIST, _EMBED_DIM)


# speedup vs baseline: 1.8749x; 1.0236x over previous
import functools

import jax
import jax.numpy as jnp
from jax import lax
from jax.experimental import pallas as pl
from jax.experimental.pallas import tpu as pltpu
from jax.experimental.pallas import tpu_sc as plsc

_WORD_NUM = 1000000
_EMBED_DIM = 64
_BATCH = 16384
_HIST = 50
_B = _BATCH * _HIST

_info = plsc.get_sparse_core_info()
_NC = _info.num_cores
_NS = _info.num_subcores
_NW = _NC * _NS
_BPW = _B // _NW
_C = 256
_G = _BPW // _C
_NBUF = 4

_mesh = plsc.VectorSubcoreMesh(core_axis_name="c", subcore_axis_name="s")


@functools.partial(
    pl.kernel,
    mesh=_mesh,
    out_type=jax.ShapeDtypeStruct((_B, _EMBED_DIM), jnp.float32),
    compiler_params=pltpu.CompilerParams(use_tc_tiling_on_sc=False),
    scratch_types=(
        [pltpu.VMEM((_G, _C), jnp.int32),
         pltpu.VMEM((_NBUF, _C, _EMBED_DIM), jnp.float32)]
        + [pltpu.SemaphoreType.DMA] * (2 * _NBUF)
    ),
)
def _sc_gather(table_hbm, idx_hbm, out_hbm, idx_v, rows_v, *sems):
    sem_g = sems[:_NBUF]
    sem_o = sems[_NBUF:]
    wid = lax.axis_index("s") * _NC + lax.axis_index("c")
    base = wid * _BPW
    pltpu.sync_copy(idx_hbm.at[wid], idx_v)

    def start_gather(g, b):
        pltpu.async_copy(table_hbm.at[idx_v.at[g]], rows_v.at[b], sem_g[b])

    def wait_gather(g, b):
        pltpu.make_async_copy(table_hbm.at[idx_v.at[g]], rows_v.at[b],
                              sem_g[b]).wait()

    def out_slice(g):
        return out_hbm.at[pl.ds(base + g * _C, _C)]

    for b in range(_NBUF):
        start_gather(b, b)

    def super_body(s, carry):
        for b in range(_NBUF):
            g = s * _NBUF + b
            wait_gather(g, b)
            pltpu.async_copy(rows_v.at[b], out_slice(g), sem_o[b])

            @pl.when(g + _NBUF < _G)
            def _():
                pltpu.make_async_copy(rows_v.at[b], out_slice(g),
                                      sem_o[b]).wait()
                start_gather(g + _NBUF, b)

        return carry

    lax.fori_loop(0, _G // _NBUF, super_body, 0)

    for b in range(_NBUF):
        g = _G - _NBUF + b
        pltpu.make_async_copy(rows_v.at[b], out_slice(g), sem_o[b]).wait()


def kernel(input_x, weight):
    idx = input_x.reshape(_NW, _G, _C).astype(jnp.int32)
    out = _sc_gather(weight, idx)
    return out.reshape(_BATCH, _HIST, _EMBED_DIM)
